# baseline (speedup 1.0000x reference)
import jax
import jax.numpy as jnp
from jax import lax
from jax.experimental import pallas as pl
from jax.experimental.pallas import tpu as pltpu
from jax.experimental.pallas import tpu_sc as plsc

BATCH = 4
SEQ_LEN = 8192
D_MODEL = 768
N_ROWS = BATCH * SEQ_LEN

NUM_CORES = 2
NUM_SUBCORES = 16
NUM_WORKERS = NUM_CORES * NUM_SUBCORES
ROWS_PER_WORKER = N_ROWS // NUM_WORKERS
CHUNK = 16
N_CHUNKS = ROWS_PER_WORKER // CHUNK
NBUF = 4
N_GROUPS = N_CHUNKS // NBUF


def _pos_enc_body(x_hbm, idx_hbm, table_hbm, out_hbm, idx_v, bufx, bufr,
                  semx, semr, semo):
    wid = lax.axis_index("s") * NUM_CORES + lax.axis_index("c")
    base = wid * ROWS_PER_WORKER
    pltpu.sync_copy(idx_hbm.at[pl.ds(base, ROWS_PER_WORKER)], idx_v)

    def issue_x(c, b):
        pltpu.async_copy(
            x_hbm.at[pl.ds(base + c * CHUNK, CHUNK)], bufx.at[b], semx.at[b]
        )

    def issue_gather(c, b):
        pltpu.async_copy(
            table_hbm.at[idx_v.at[pl.ds(c * CHUNK, CHUNK)]],
            bufr.at[b],
            semr.at[b],
        )

    def issue_store(c, b):
        pltpu.async_copy(
            bufx.at[b], out_hbm.at[pl.ds(base + c * CHUNK, CHUNK)], semo.at[b]
        )

    def wait_x(b):
        pltpu.make_async_copy(
            x_hbm.at[pl.ds(0, CHUNK)], bufx.at[b], semx.at[b]
        ).wait()

    def wait_r(b):
        pltpu.make_async_copy(
            x_hbm.at[pl.ds(0, CHUNK)], bufr.at[b], semr.at[b]
        ).wait()

    def wait_o(b):
        pltpu.make_async_copy(
            bufx.at[b], out_hbm.at[pl.ds(0, CHUNK)], semo.at[b]
        ).wait()

    def alu(b):
        def row_body(r, rcarry):
            for j in range(D_MODEL // 16):
                s = pl.ds(j * 16, 16)
                bufx[b, r, s] = bufx[b, r, s] + bufr[b, r, s]
            return rcarry

        lax.fori_loop(0, CHUNK, row_body, 0)

    def chunk_body(c, b, issue_next_gather, first_group):
        wait_r(b)
        wait_x(b)
        alu(b)
        issue_store(c, b)
        if issue_next_gather:
            issue_gather(c + NBUF, b)
        last_chunk = (not issue_next_gather) and b == NBUF - 1
        if not last_chunk:
            nb = (b + 1) % NBUF
            if not (first_group and b < NBUF - 1):
                wait_o(nb)
            issue_x(c + 1, nb)

    issue_x(0, 0)
    for b in range(NBUF):
        issue_gather(b, b)

    for b in range(NBUF):
        chunk_body(b, b, True, True)

    def group_body(g, carry):
        c0 = g * NBUF
        for b in range(NBUF):
            chunk_body(c0 + b, b, True, False)
        return carry

    lax.fori_loop(1, N_GROUPS - 1, group_body, 0)

    c0 = (N_GROUPS - 1) * NBUF
    for b in range(NBUF):
        chunk_body(c0 + b, b, False, False)

    for b in range(NBUF):
        wait_o(b)


@jax.jit
def kernel(x, position_ids, pos_table):
    x2 = x.reshape(N_ROWS, D_MODEL)
    idx = position_ids.astype(jnp.int32).reshape(N_ROWS)

    mesh = plsc.VectorSubcoreMesh(
        core_axis_name="c",
        subcore_axis_name="s",
        num_cores=NUM_CORES,
        num_subcores=NUM_SUBCORES,
    )
    out = pl.kernel(
        _pos_enc_body,
        out_type=jax.ShapeDtypeStruct((N_ROWS, D_MODEL), jnp.float32),
        mesh=mesh,
        scratch_types=[
            pltpu.VMEM((ROWS_PER_WORKER,), jnp.int32),
            pltpu.VMEM((NBUF, CHUNK, D_MODEL), jnp.float32),
            pltpu.VMEM((NBUF, CHUNK, D_MODEL), jnp.float32),
            pltpu.SemaphoreType.DMA((NBUF,)),
            pltpu.SemaphoreType.DMA((NBUF,)),
            pltpu.SemaphoreType.DMA((NBUF,)),
        ],
    )(x2, idx, pos_table)
    return out.reshape(BATCH, SEQ_LEN, D_MODEL)


# speedup vs baseline: 1.3533x; 1.3533x over previous
import jax
import jax.numpy as jnp
from jax import lax
from jax.experimental import pallas as pl
from jax.experimental.pallas import tpu as pltpu
from jax.experimental.pallas import tpu_sc as plsc

BATCH = 4
SEQ_LEN = 8192
D_MODEL = 768
N_ROWS = BATCH * SEQ_LEN

NUM_CORES = 2
NUM_SUBCORES = 16
NUM_WORKERS = NUM_CORES * NUM_SUBCORES
ROWS_PER_WORKER = N_ROWS // NUM_WORKERS
CHUNK = 32
N_CHUNKS = ROWS_PER_WORKER // CHUNK
NX = 3
NR = 2
PERIOD = 6


def _pos_enc_body(x_hbm, idx_hbm, table_hbm, out_hbm, idx_v, bufx, bufr,
                  semx, semr, semo):
    wid = lax.axis_index("s") * NUM_CORES + lax.axis_index("c")
    base = wid * ROWS_PER_WORKER
    pltpu.sync_copy(idx_hbm.at[pl.ds(base, ROWS_PER_WORKER)], idx_v)

    def issue_x(c, xs):
        pltpu.async_copy(
            x_hbm.at[pl.ds(base + c * CHUNK, CHUNK)], bufx.at[xs], semx.at[xs]
        )

    def issue_gather(c, rs):
        pltpu.async_copy(
            table_hbm.at[idx_v.at[pl.ds(c * CHUNK, CHUNK)]],
            bufr.at[rs],
            semr.at[rs],
        )

    def issue_store(c, xs):
        pltpu.async_copy(
            bufx.at[xs], out_hbm.at[pl.ds(base + c * CHUNK, CHUNK)], semo.at[xs]
        )

    def wait_x(xs):
        pltpu.make_async_copy(
            x_hbm.at[pl.ds(0, CHUNK)], bufx.at[xs], semx.at[xs]
        ).wait()

    def wait_r(rs):
        pltpu.make_async_copy(
            x_hbm.at[pl.ds(0, CHUNK)], bufr.at[rs], semr.at[rs]
        ).wait()

    def wait_o(xs):
        pltpu.make_async_copy(
            bufx.at[xs], out_hbm.at[pl.ds(0, CHUNK)], semo.at[xs]
        ).wait()

    def alu(xs, rs):
        def row_body(r, rcarry):
            for j in range(D_MODEL // 16):
                s = pl.ds(j * 16, 16)
                bufx[xs, r, s] = bufx[xs, r, s] + bufr[rs, r, s]
            return rcarry

        lax.fori_loop(0, CHUNK, row_body, 0)

    def chunk_body(c, i, drain_store, more_x, more_gather):
        xs = i % NX
        rs = i % NR
        if more_x:
            nxs = (i + 1) % NX
            if drain_store:
                wait_o(nxs)
            issue_x(c + 1, nxs)
        wait_r(rs)
        wait_x(xs)
        alu(xs, rs)
        issue_store(c, xs)
        if more_gather:
            issue_gather(c + 2, rs)

    issue_x(0, 0)
    issue_gather(0, 0)
    issue_gather(1, 1)

    for i in range(PERIOD):
        chunk_body(i, i, i >= 2, True, True)

    def period_body(g, carry):
        c0 = g * PERIOD
        for i in range(PERIOD):
            chunk_body(c0 + i, i, True, True, True)
        return carry

    lax.fori_loop(1, (N_CHUNKS - 2) // PERIOD, period_body, 0)

    chunk_body(30, 0, True, True, False)
    chunk_body(31, 1, False, False, False)

    wait_o(2)
    wait_o(0)
    wait_o(1)


@jax.jit
def kernel(x, position_ids, pos_table):
    x2 = x.reshape(N_ROWS, D_MODEL)
    idx = position_ids.astype(jnp.int32).reshape(N_ROWS)

    mesh = plsc.VectorSubcoreMesh(
        core_axis_name="c",
        subcore_axis_name="s",
        num_cores=NUM_CORES,
        num_subcores=NUM_SUBCORES,
    )
    out = pl.kernel(
        _pos_enc_body,
        out_type=jax.ShapeDtypeStruct((N_ROWS, D_MODEL), jnp.float32),
        mesh=mesh,
        scratch_types=[
            pltpu.VMEM((ROWS_PER_WORKER,), jnp.int32),
            pltpu.VMEM((NX, CHUNK, D_MODEL), jnp.float32),
            pltpu.VMEM((NR, CHUNK, D_MODEL), jnp.float32),
            pltpu.SemaphoreType.DMA((NX,)),
            pltpu.SemaphoreType.DMA((NR,)),
            pltpu.SemaphoreType.DMA((NX,)),
        ],
    )(x2, idx, pos_table)
    return out.reshape(BATCH, SEQ_LEN, D_MODEL)


# speedup vs baseline: 1.4347x; 1.0601x over previous
import jax
import jax.numpy as jnp
from jax import lax
from jax.experimental import pallas as pl
from jax.experimental.pallas import tpu as pltpu
from jax.experimental.pallas import tpu_sc as plsc

BATCH = 4
SEQ_LEN = 8192
D_MODEL = 768
N_ROWS = BATCH * SEQ_LEN

NUM_CORES = 2
NUM_SUBCORES = 16
NUM_WORKERS = NUM_CORES * NUM_SUBCORES
ROWS_PER_WORKER = N_ROWS // NUM_WORKERS
CHUNK = 64
N_CHUNKS = ROWS_PER_WORKER // CHUNK


def _pos_enc_body(x_hbm, idx_hbm, table_hbm, out_hbm, idx_v, bufx, bufr, sem):
    wid = lax.axis_index("s") * NUM_CORES + lax.axis_index("c")
    base = wid * ROWS_PER_WORKER
    pltpu.sync_copy(idx_hbm.at[pl.ds(base, ROWS_PER_WORKER)], idx_v)

    def chunk_body(c, carry):
        row0 = base + c * CHUNK
        gather = pltpu.async_copy(
            table_hbm.at[idx_v.at[pl.ds(c * CHUNK, CHUNK)]], bufr, sem
        )
        pltpu.sync_copy(x_hbm.at[pl.ds(row0, CHUNK)], bufx)
        gather.wait()

        def row_body(r, rcarry):
            for j in range(D_MODEL // 16):
                s = pl.ds(j * 16, 16)
                bufx[r, s] = bufx[r, s] + bufr[r, s]
            return rcarry

        lax.fori_loop(0, CHUNK, row_body, 0)
        pltpu.sync_copy(bufx, out_hbm.at[pl.ds(row0, CHUNK)])
        return carry

    lax.fori_loop(0, N_CHUNKS, chunk_body, 0)


@jax.jit
def kernel(x, position_ids, pos_table):
    x2 = x.reshape(N_ROWS, D_MODEL)
    idx = position_ids.astype(jnp.int32).reshape(N_ROWS)

    mesh = plsc.VectorSubcoreMesh(
        core_axis_name="c",
        subcore_axis_name="s",
        num_cores=NUM_CORES,
        num_subcores=NUM_SUBCORES,
    )
    out = pl.kernel(
        _pos_enc_body,
        out_type=jax.ShapeDtypeStruct((N_ROWS, D_MODEL), jnp.float32),
        mesh=mesh,
        scratch_types=[
            pltpu.VMEM((ROWS_PER_WORKER,), jnp.int32),
            pltpu.VMEM((CHUNK, D_MODEL), jnp.float32),
            pltpu.VMEM((CHUNK, D_MODEL), jnp.float32),
            pltpu.SemaphoreType.DMA,
        ],
    )(x2, idx, pos_table)
    return out.reshape(BATCH, SEQ_LEN, D_MODEL)


# speedup vs baseline: 1.5024x; 1.0472x over previous
import jax
import jax.numpy as jnp
from jax import lax
from jax.experimental import pallas as pl
from jax.experimental.pallas import tpu as pltpu
from jax.experimental.pallas import tpu_sc as plsc

BATCH = 4
SEQ_LEN = 8192
D_MODEL = 768
N_ROWS = BATCH * SEQ_LEN

NUM_CORES = 2
NUM_SUBCORES = 16
NUM_WORKERS = NUM_CORES * NUM_SUBCORES
ROWS_PER_WORKER = N_ROWS // NUM_WORKERS
CHUNK = 64
N_CHUNKS = ROWS_PER_WORKER // CHUNK


def _pos_enc_body(x_hbm, idx_hbm, table_hbm, out_hbm, idx_v, bufx, bufr,
                  semg, semo):
    wid = lax.axis_index("s") * NUM_CORES + lax.axis_index("c")
    base = wid * ROWS_PER_WORKER
    pltpu.sync_copy(idx_hbm.at[pl.ds(base, ROWS_PER_WORKER)], idx_v)

    def alu():
        def row_body(r, rcarry):
            for j in range(D_MODEL // 16):
                s = pl.ds(j * 16, 16)
                bufx[r, s] = bufx[r, s] + bufr[r, s]
            return rcarry

        lax.fori_loop(0, CHUNK, row_body, 0)

    def chunk_work(c, drain_prev_store):
        row0 = base + c * CHUNK
        gather = pltpu.async_copy(
            table_hbm.at[idx_v.at[pl.ds(c * CHUNK, CHUNK)]], bufr, semg
        )
        if drain_prev_store:
            pltpu.make_async_copy(
                bufx, out_hbm.at[pl.ds(0, CHUNK)], semo
            ).wait()
        pltpu.sync_copy(x_hbm.at[pl.ds(row0, CHUNK)], bufx)
        gather.wait()
        alu()
        pltpu.async_copy(bufx, out_hbm.at[pl.ds(row0, CHUNK)], semo)

    chunk_work(0, False)

    def chunk_body(c, carry):
        chunk_work(c, True)
        return carry

    lax.fori_loop(1, N_CHUNKS, chunk_body, 0)
    pltpu.make_async_copy(bufx, out_hbm.at[pl.ds(0, CHUNK)], semo).wait()


@jax.jit
def kernel(x, position_ids, pos_table):
    x2 = x.reshape(N_ROWS, D_MODEL)
    idx = position_ids.astype(jnp.int32).reshape(N_ROWS)

    mesh = plsc.VectorSubcoreMesh(
        core_axis_name="c",
        subcore_axis_name="s",
        num_cores=NUM_CORES,
        num_subcores=NUM_SUBCORES,
    )
    out = pl.kernel(
        _pos_enc_body,
        out_type=jax.ShapeDtypeStruct((N_ROWS, D_MODEL), jnp.float32),
        mesh=mesh,
        scratch_types=[
            pltpu.VMEM((ROWS_PER_WORKER,), jnp.int32),
            pltpu.VMEM((CHUNK, D_MODEL), jnp.float32),
            pltpu.VMEM((CHUNK, D_MODEL), jnp.float32),
            pltpu.SemaphoreType.DMA,
            pltpu.SemaphoreType.DMA,
        ],
    )(x2, idx, pos_table)
    return out.reshape(BATCH, SEQ_LEN, D_MODEL)
